# baseline (speedup 1.0000x reference)
import functools

import jax
import jax.numpy as jnp
from jax import lax
from jax.experimental import pallas as pl
from jax.experimental.pallas import tpu as pltpu
from jax.experimental.pallas import tpu_sc as plsc

D_MODEL = 1024

_info = plsc.get_sparse_core_info()
_NC = _info.num_cores
_NS = _info.num_subcores
_NW = _NC * _NS

_N = 4 * 4096
_PER_W = _N // _NW
_NSEM = 16

_mesh = plsc.VectorSubcoreMesh(core_axis_name="c", subcore_axis_name="s")


@functools.partial(
    pl.kernel,
    mesh=_mesh,
    out_type=jax.ShapeDtypeStruct((_N, D_MODEL), jnp.float32),
    scratch_types=[
        pltpu.VMEM((_PER_W,), jnp.int32),
    ] + [pltpu.SemaphoreType.DMA] * _NSEM,
)
def _pe_gather(table_hbm, idx_hbm, out_hbm, idx_v, *sems):
    wid = lax.axis_index("s") * _NC + lax.axis_index("c")
    base = wid * _PER_W
    pltpu.sync_copy(idx_hbm.at[wid], idx_v)

    def row_dma(row, i, k):
        return pltpu.make_async_copy(
            table_hbm.at[pl.ds(row, 1)],
            out_hbm.at[pl.ds(base + i, 1)], sems[k])

    vec0 = idx_v[pl.ds(0, _NSEM)]
    for k in range(_NSEM):
        row_dma(vec0[k], k, k).start()

    def outer(o, carry):
        vec = idx_v[pl.ds((o + 1) * _NSEM, _NSEM)]
        for k in range(_NSEM):
            i = (o + 1) * _NSEM + k
            row_dma(vec[k], i, k).wait()
            row_dma(vec[k], i, k).start()
        return carry

    lax.fori_loop(0, _PER_W // _NSEM - 1, outer, 0)
    for k in range(_NSEM):
        row_dma(0, k, k).wait()


def kernel(x, position_ids, pe):
    del x
    batch, seq_len = position_ids.shape
    table = pe.reshape(pe.shape[1], D_MODEL)
    idx = position_ids.reshape(_NW, _PER_W).astype(jnp.int32)
    out = _pe_gather(table, idx)
    return out.reshape(batch, seq_len, D_MODEL)


# speedup vs baseline: 29.4225x; 29.4225x over previous
import functools

import jax
import jax.numpy as jnp
from jax import lax
from jax.experimental import pallas as pl
from jax.experimental.pallas import tpu as pltpu
from jax.experimental.pallas import tpu_sc as plsc

D_MODEL = 1024

_info = plsc.get_sparse_core_info()
_NC = _info.num_cores
_NS = _info.num_subcores
_NW = _NC * _NS

_N = 4 * 4096
_PER_W = _N // _NW
_C = 32
_N_CHUNKS = _PER_W // _C
_NBUF = 3
_SUB = 8
_NSUB = _C // _SUB
_NSTG = 3

_mesh = plsc.VectorSubcoreMesh(core_axis_name="c", subcore_axis_name="s")


@functools.partial(
    pl.kernel,
    mesh=_mesh,
    out_type=jax.ShapeDtypeStruct((_N, D_MODEL), jnp.float32),
    scratch_types=[
        pltpu.VMEM((_N_CHUNKS, _C), jnp.int32),
        pltpu.VMEM_SHARED((_NS, _NSTG, _SUB, D_MODEL), jnp.float32),
    ] + [pltpu.VMEM((_C, D_MODEL), jnp.float32)] * _NBUF
      + [pltpu.SemaphoreType.DMA] * _NBUF
      + [pltpu.SemaphoreType.DMA] * _NSTG
      + [pltpu.SemaphoreType.DMA] * _NSTG,
)
def _pe_gather(table_hbm, idx_hbm, out_hbm, idx_v, stage, *bufs_sems):
    rows = bufs_sems[:_NBUF]
    gsems = bufs_sems[_NBUF:2 * _NBUF]
    xsems = bufs_sems[2 * _NBUF:2 * _NBUF + _NSTG]
    esems = bufs_sems[2 * _NBUF + _NSTG:]
    sid = lax.axis_index("s")
    wid = sid * _NC + lax.axis_index("c")
    base = wid * _PER_W
    pltpu.sync_copy(idx_hbm.at[wid], idx_v)
    gathers = [None] * _NBUF
    egress = [None] * _NSTG
    for k in range(_NBUF - 1):
        gathers[k] = pltpu.async_copy(
            table_hbm.at[idx_v.at[k]], rows[k], gsems[k])
    sub = 0
    for i in range(_N_CHUNKS):
        b = i % _NBUF
        j = i + _NBUF - 1
        if j < _N_CHUNKS:
            bj = j % _NBUF
            gathers[bj] = pltpu.async_copy(
                table_hbm.at[idx_v.at[j]], rows[bj], gsems[bj])
        gathers[b].wait()
        for s in range(_NSUB):
            r = sub % _NSTG
            if egress[r] is not None:
                egress[r].wait()
            pltpu.async_copy(
                rows[b].at[pl.ds(s * _SUB, _SUB)], stage.at[sid, r],
                xsems[r]).wait()
            egress[r] = pltpu.async_copy(
                stage.at[sid, r], out_hbm.at[pl.ds(base + sub * _SUB, _SUB)],
                esems[r])
            sub += 1
    for r in range(_NSTG):
        if egress[r] is not None:
            egress[r].wait()


def kernel(x, position_ids, pe):
    del x
    batch, seq_len = position_ids.shape
    table = pe.reshape(pe.shape[1], D_MODEL)
    idx = position_ids.reshape(_NW, _N_CHUNKS, _C).astype(jnp.int32)
    out = _pe_gather(table, idx)
    return out.reshape(batch, seq_len, D_MODEL)


# speedup vs baseline: 30.5875x; 1.0396x over previous
import functools

import jax
import jax.numpy as jnp
from jax import lax
from jax.experimental import pallas as pl
from jax.experimental.pallas import tpu as pltpu
from jax.experimental.pallas import tpu_sc as plsc

D_MODEL = 1024

_info = plsc.get_sparse_core_info()
_NC = _info.num_cores
_NS = _info.num_subcores
_NW = _NC * _NS

_N = 4 * 4096
_PER_W = _N // _NW
_C = 16
_N_CHUNKS = _PER_W // _C
_NBUF = 6

_mesh = plsc.VectorSubcoreMesh(core_axis_name="c", subcore_axis_name="s")


@functools.partial(
    pl.kernel,
    mesh=_mesh,
    out_type=jax.ShapeDtypeStruct((_N, D_MODEL), jnp.float32),
    scratch_types=[
        pltpu.VMEM((_N_CHUNKS, _C), jnp.int32),
    ] + [pltpu.VMEM((_C, D_MODEL), jnp.float32)] * _NBUF
      + [pltpu.SemaphoreType.DMA] * (2 * _NBUF),
)
def _pe_gather(table_hbm, idx_hbm, out_hbm, idx_v, *bufs_sems):
    rows = bufs_sems[:_NBUF]
    gsems = bufs_sems[_NBUF:2 * _NBUF]
    ssems = bufs_sems[2 * _NBUF:]
    wid = lax.axis_index("s") * _NC + lax.axis_index("c")
    base = wid * _PER_W
    pltpu.sync_copy(idx_hbm.at[wid], idx_v)
    gathers = [None] * _NBUF
    stores = [None] * _NBUF
    for k in range(_NBUF - 1):
        gathers[k] = pltpu.async_copy(
            table_hbm.at[idx_v.at[k]], rows[k], gsems[k])
    for i in range(_N_CHUNKS):
        b = i % _NBUF
        j = i + _NBUF - 1
        if j < _N_CHUNKS:
            bj = j % _NBUF
            if stores[bj] is not None:
                stores[bj].wait()
            gathers[bj] = pltpu.async_copy(
                table_hbm.at[idx_v.at[j]], rows[bj], gsems[bj])
        gathers[b].wait()
        stores[b] = pltpu.async_copy(
            rows[b], out_hbm.at[pl.ds(base + i * _C, _C)], ssems[b])
    for i in range(_N_CHUNKS - _NBUF, _N_CHUNKS):
        stores[i % _NBUF].wait()


def kernel(x, position_ids, pe):
    del x
    batch, seq_len = position_ids.shape
    table = pe.reshape(pe.shape[1], D_MODEL)
    idx = position_ids.reshape(_NW, _N_CHUNKS, _C).astype(jnp.int32)
    out = _pe_gather(table, idx)
    return out.reshape(batch, seq_len, D_MODEL)
